# baseline (speedup 1.0000x reference)
import jax
import jax.numpy as jnp
from jax.experimental import pallas as pl
from jax.experimental.pallas import tpu as pltpu

S, D = 2048, 1024
P1, P2 = 256, 64


def _proj_body(x_ref, p_ref, o_ref):
    o_ref[...] = jnp.dot(
        x_ref[...].astype(jnp.bfloat16),
        p_ref[...].astype(jnp.bfloat16),
        preferred_element_type=jnp.float32,
    )


def _matmul_body(a_ref, w_ref, b_ref, o_ref):
    acc = jnp.dot(
        a_ref[...].astype(jnp.bfloat16),
        w_ref[...].astype(jnp.bfloat16),
        preferred_element_type=jnp.float32,
    )
    o_ref[...] = acc + b_ref[...]


def _tiled_matmul(a, w, b2d, tn):
    k = a.shape[1]
    n = w.shape[1]
    grid = pl.cdiv(n, tn)
    return pl.pallas_call(
        _matmul_body,
        grid=(grid,),
        in_specs=[
            pl.BlockSpec((S, k), lambda j: (0, 0)),
            pl.BlockSpec((k, tn), lambda j: (0, j)),
            pl.BlockSpec((1, tn), lambda j: (0, j)),
        ],
        out_specs=pl.BlockSpec((S, tn), lambda j: (0, j)),
        out_shape=jax.ShapeDtypeStruct((S, n), jnp.float32),
        compiler_params=pltpu.CompilerParams(
            dimension_semantics=("parallel",),
        ),
    )(a, w, b2d)


def kernel(x, proj1, proj2, W0, W1, W2, b0, b1, b2):
    x2 = x.reshape(S, D)
    projc = jnp.concatenate([proj1, proj2], axis=1)
    xp = pl.pallas_call(
        _proj_body,
        out_shape=jax.ShapeDtypeStruct((S, P1 + P2), jnp.float32),
    )(x2, projc)
    xp1 = xp[:, :P1]
    xp2 = xp[:, P1:]

    head = _tiled_matmul(x2, W0, b0.reshape(1, -1), 1024)
    c1 = _tiled_matmul(xp1, W1, b1.reshape(1, -1), 2048)
    c2 = _tiled_matmul(xp2, W2, b2.reshape(1, -1), 2048)

    n0, n1, n2 = W0.shape[1], W1.shape[1], W2.shape[1]
    return (head.reshape(1, S, n0), c1.reshape(1, S, n1), c2.reshape(1, S, n2))
